# baseline (speedup 1.0000x reference)
import functools

import jax
import jax.numpy as jnp
from jax import lax
from jax.experimental import pallas as pl
from jax.experimental.pallas import tpu as pltpu
from jax.experimental.pallas import tpu_sc as plsc

N = 10000
E = 320000
G = 16
D = 128
NCLASSES_OUT = 10
NC = 2
NS = 16
NW = NC * NS
NPAD = 10240
SLAB = NPAD // NS
CHUNK = 128
KCH = 80
MBLK = 8
EPW = KCH * CHUNK
EPAD = NW * EPW
NPADROWS = 240

def _deg_body(c2_hbm, ones_hbm, zrow_hbm, out_hbm, cidx, ones_v, deg_sh):
    cc = lax.axis_index("c")
    ss = lax.axis_index("s")
    pltpu.sync_copy(zrow_hbm, deg_sh.at[pl.ds(ss * SLAB, SLAB)])
    pltpu.sync_copy(ones_hbm, ones_v)
    brow = (cc * NS + ss) * KCH
    pltpu.sync_copy(c2_hbm.at[pl.ds(brow, KCH)], cidx)
    plsc.subcore_barrier()

    def body(i, carry):
        pltpu.sync_copy(ones_v, deg_sh.at[cidx.at[i]], add=True)
        return carry

    lax.fori_loop(0, KCH, body, 0)
    plsc.subcore_barrier()
    pltpu.sync_copy(deg_sh.at[pl.ds(ss * SLAB, SLAB)],
                    out_hbm.at[cc, pl.ds(ss * SLAB, SLAB)])


def _mp_body(src_hbm, r2_hbm, c2_hbm, zslab_hbm, out_hbm,
             ridx, cblk0, cblk1, rows0, rows1, acc_sh, sem0, sem1):
    cc = lax.axis_index("c")
    ss = lax.axis_index("s")
    pltpu.sync_copy(zslab_hbm, acc_sh.at[pl.ds(ss * SLAB, SLAB)])
    brow = (cc * NS + ss) * KCH
    pltpu.sync_copy(r2_hbm.at[pl.ds(brow, KCH)], ridx)
    plsc.subcore_barrier()

    pltpu.async_copy(src_hbm.at[ridx.at[0]], rows0, sem0)

    def block(m, cblk, last):
        pltpu.sync_copy(c2_hbm.at[pl.ds(brow + MBLK * m, MBLK)], cblk)
        for q in range(MBLK // 2):
            a = MBLK * m + 2 * q + 1
            pltpu.async_copy(src_hbm.at[ridx.at[a]], rows1, sem1)
            pltpu.make_async_copy(src_hbm.at[ridx.at[0]], rows0, sem0).wait()
            pltpu.sync_copy(rows0, acc_sh.at[cblk.at[2 * q]], add=True)
            if not (last and q == MBLK // 2 - 1):
                pltpu.async_copy(src_hbm.at[ridx.at[a + 1]], rows0, sem0)
            pltpu.make_async_copy(src_hbm.at[ridx.at[0]], rows1, sem1).wait()
            pltpu.sync_copy(rows1, acc_sh.at[cblk.at[2 * q + 1]], add=True)

    def body(p, carry):
        block(2 * p, cblk0, False)
        block(2 * p + 1, cblk1, False)
        return carry

    nblk = KCH // MBLK
    lax.fori_loop(0, nblk // 2 - 1, body, 0)
    block(nblk - 2, cblk0, False)
    block(nblk - 1, cblk1, True)
    plsc.subcore_barrier()
    pltpu.sync_copy(acc_sh.at[pl.ds(ss * SLAB, SLAB)],
                    out_hbm.at[cc, pl.ds(ss * SLAB, SLAB)])


def _bn(x, g, b):
    m = jnp.mean(x, axis=0)
    v = jnp.var(x, axis=0)
    return (x - m) / jnp.sqrt(v + 1e-5) * g + b


def _dis_from(deg2_ref):
    deg = deg2_ref[0, :N] + deg2_ref[1, :N] + 1.0
    return lax.rsqrt(deg)


def _dense1_body(x0_ref, w_ref, b_ref, g_ref, be_ref, cw_ref, deg2_ref, out_ref):
    h = jnp.maximum(jnp.dot(x0_ref[...], w_ref[...],
                            preferred_element_type=jnp.float32) + b_ref[...], 0.0)
    h = _bn(h, g_ref[...], be_ref[...])
    gmat = jnp.dot(h, cw_ref[...], preferred_element_type=jnp.float32)
    out_ref[:N] = _dis_from(deg2_ref)[:, None] * gmat
    out_ref[N:] = jnp.zeros((NPAD - N, D), jnp.float32)


def _dense2_body(acc_ref, gp_ref, deg2_ref, cb_ref, ng_ref, nb_ref,
                 w_ref, b_ref, g_ref, be_ref, cw_ref, out_ref):
    dis = _dis_from(deg2_ref)
    x = dis[:, None] * (acc_ref[0, :N] + acc_ref[1, :N] + gp_ref[:N]) + cb_ref[...]
    x = jnp.maximum(_bn(x, ng_ref[...], nb_ref[...]), 0.0)
    h = jnp.maximum(jnp.dot(x, w_ref[...],
                            preferred_element_type=jnp.float32) + b_ref[...], 0.0)
    h = _bn(h, g_ref[...], be_ref[...])
    gmat = jnp.dot(h, cw_ref[...], preferred_element_type=jnp.float32)
    out_ref[:N] = dis[:, None] * gmat
    out_ref[N:] = jnp.zeros((NPAD - N, D), jnp.float32)


def _dense3_body(acc_ref, gp_ref, deg2_ref, cb_ref, ng_ref, nb_ref, batch_ref,
                 f1w_ref, f1b_ref, f2w_ref, f2b_ref, out_ref):
    dis = _dis_from(deg2_ref)
    x = dis[:, None] * (acc_ref[0, :N] + acc_ref[1, :N] + gp_ref[:N]) + cb_ref[...]
    x = jnp.maximum(_bn(x, ng_ref[...], nb_ref[...]), 0.0)
    onehot = (batch_ref[...][:, None]
              == lax.broadcasted_iota(jnp.int32, (N, G), 1)).astype(jnp.float32)
    sums = lax.dot_general(onehot, x, (((0,), (0,)), ((), ())),
                           preferred_element_type=jnp.float32)
    cnt = jnp.sum(onehot, axis=0)
    pooled = sums / jnp.maximum(cnt, 1.0)[:, None]
    y = jnp.maximum(jnp.dot(pooled, f1w_ref[...],
                            preferred_element_type=jnp.float32) + f1b_ref[...], 0.0)
    out_ref[...] = jnp.dot(y, f2w_ref[...],
                           preferred_element_type=jnp.float32) + f2b_ref[...]


def _tc_call(body, out_shape, *args):
    return pl.pallas_call(body, out_shape=out_shape)(*args)


@functools.lru_cache(maxsize=None)
def _sc_kernels():
    mesh = plsc.VectorSubcoreMesh(core_axis_name="c", subcore_axis_name="s",
                                  num_cores=NC, num_subcores=NS)
    params = pltpu.CompilerParams()
    deg_k = pl.kernel(
        _deg_body,
        out_type=jax.ShapeDtypeStruct((NC, NPAD), jnp.float32),
        mesh=mesh,
        compiler_params=params,
        scratch_types=[
            pltpu.VMEM((KCH, CHUNK), jnp.int32),
            pltpu.VMEM((CHUNK,), jnp.float32),
            pltpu.VMEM_SHARED((NPAD,), jnp.float32),
        ],
    )
    mp_k = pl.kernel(
        _mp_body,
        out_type=jax.ShapeDtypeStruct((NC, NPAD, D), jnp.float32),
        mesh=mesh,
        compiler_params=params,
        scratch_types=[
            pltpu.VMEM((KCH, CHUNK), jnp.int32),
            pltpu.VMEM((MBLK, CHUNK), jnp.int32),
            pltpu.VMEM((MBLK, CHUNK), jnp.int32),
            pltpu.VMEM((CHUNK, D), jnp.float32),
            pltpu.VMEM((CHUNK, D), jnp.float32),
            pltpu.VMEM_SHARED((NPAD, D), jnp.float32),
            pltpu.SemaphoreType.DMA,
            pltpu.SemaphoreType.DMA,
        ],
    )
    return deg_k, mp_k


def kernel(x0, edge_index, batch,
           mlp0_W, mlp0_b, mlp0_gamma, mlp0_beta,
           conv0_W, conv0_b, norm0_gamma, norm0_beta,
           mlp1_W, mlp1_b, mlp1_gamma, mlp1_beta,
           conv1_W, conv1_b, norm1_gamma, norm1_beta,
           fc1_W, fc1_b, fc2_W, fc2_b):
    padv = N + (jnp.arange(EPAD - E, dtype=jnp.int32) % NPADROWS)
    r2 = jnp.concatenate([edge_index[0], padv]).reshape(EPAD // CHUNK, CHUNK)
    c2 = jnp.concatenate([edge_index[1], padv]).reshape(EPAD // CHUNK, CHUNK)
    ones_chunk = jnp.ones((CHUNK,), jnp.float32)
    zrow = jnp.zeros((SLAB,), jnp.float32)
    zslab = jnp.zeros((SLAB, D), jnp.float32)

    deg_kernel, mp_kernel = _sc_kernels()
    deg2 = deg_kernel(c2, ones_chunk, zrow)

    g0p = _tc_call(_dense1_body, jax.ShapeDtypeStruct((NPAD, D), jnp.float32),
                   x0, mlp0_W, mlp0_b, mlp0_gamma, mlp0_beta, conv0_W, deg2)
    acc0 = mp_kernel(g0p, r2, c2, zslab)
    g1p = _tc_call(_dense2_body, jax.ShapeDtypeStruct((NPAD, D), jnp.float32),
                   acc0, g0p, deg2, conv0_b, norm0_gamma, norm0_beta,
                   mlp1_W, mlp1_b, mlp1_gamma, mlp1_beta, conv1_W)
    acc1 = mp_kernel(g1p, r2, c2, zslab)
    y = _tc_call(_dense3_body, jax.ShapeDtypeStruct((G, NCLASSES_OUT), jnp.float32),
                 acc1, g1p, deg2, conv1_b, norm1_gamma, norm1_beta, batch,
                 fc1_W, fc1_b, fc2_W, fc2_b)
    return (y, y)


# speedup vs baseline: 1.0147x; 1.0147x over previous
import functools

import jax
import jax.numpy as jnp
from jax import lax
from jax.experimental import pallas as pl
from jax.experimental.pallas import tpu as pltpu
from jax.experimental.pallas import tpu_sc as plsc

N = 10000
E = 320000
G = 16
D = 128
NCLASSES_OUT = 10
NC = 2
NS = 16
NW = NC * NS
NPAD = 10240
SLAB = NPAD // NS
CHUNK = 128
KCH = 80
MBLK = 8
EPW = KCH * CHUNK
EPAD = NW * EPW
NPADROWS = 240

def _deg_body(c2_hbm, ones_hbm, zrow_hbm, out_hbm, cidx, ones_v, deg_sh, dsem):
    cc = lax.axis_index("c")
    ss = lax.axis_index("s")
    pltpu.sync_copy(zrow_hbm, deg_sh.at[pl.ds(ss * SLAB, SLAB)])
    pltpu.sync_copy(ones_hbm, ones_v)
    brow = (cc * NS + ss) * KCH
    pltpu.sync_copy(c2_hbm.at[pl.ds(brow, KCH)], cidx)
    plsc.subcore_barrier()

    def body(m, carry):
        for k in range(MBLK):
            pltpu.async_copy(ones_v, deg_sh.at[cidx.at[MBLK * m + k]], dsem,
                             add=True)
        for k in range(MBLK):
            pltpu.make_async_copy(ones_v, deg_sh.at[cidx.at[0]], dsem).wait()
        return carry

    lax.fori_loop(0, KCH // MBLK, body, 0)
    plsc.subcore_barrier()
    pltpu.sync_copy(deg_sh.at[pl.ds(ss * SLAB, SLAB)],
                    out_hbm.at[cc, pl.ds(ss * SLAB, SLAB)])


def _mp_body(src_hbm, r2_hbm, c2_hbm, zslab_hbm, out_hbm,
             ridx, cblk0, cblk1, rows0, rows1, acc_sh, sem0, sem1):
    cc = lax.axis_index("c")
    ss = lax.axis_index("s")
    pltpu.sync_copy(zslab_hbm, acc_sh.at[pl.ds(ss * SLAB, SLAB)])
    brow = (cc * NS + ss) * KCH
    pltpu.sync_copy(r2_hbm.at[pl.ds(brow, KCH)], ridx)
    plsc.subcore_barrier()

    pltpu.async_copy(src_hbm.at[ridx.at[0]], rows0, sem0)

    def block(m, cblk, last):
        pltpu.sync_copy(c2_hbm.at[pl.ds(brow + MBLK * m, MBLK)], cblk)
        for q in range(MBLK // 2):
            a = MBLK * m + 2 * q + 1
            pltpu.async_copy(src_hbm.at[ridx.at[a]], rows1, sem1)
            pltpu.make_async_copy(src_hbm.at[ridx.at[0]], rows0, sem0).wait()
            pltpu.sync_copy(rows0, acc_sh.at[cblk.at[2 * q]], add=True)
            if not (last and q == MBLK // 2 - 1):
                pltpu.async_copy(src_hbm.at[ridx.at[a + 1]], rows0, sem0)
            pltpu.make_async_copy(src_hbm.at[ridx.at[0]], rows1, sem1).wait()
            pltpu.sync_copy(rows1, acc_sh.at[cblk.at[2 * q + 1]], add=True)

    def body(p, carry):
        block(2 * p, cblk0, False)
        block(2 * p + 1, cblk1, False)
        return carry

    nblk = KCH // MBLK
    lax.fori_loop(0, nblk // 2 - 1, body, 0)
    block(nblk - 2, cblk0, False)
    block(nblk - 1, cblk1, True)
    plsc.subcore_barrier()
    pltpu.sync_copy(acc_sh.at[pl.ds(ss * SLAB, SLAB)],
                    out_hbm.at[cc, pl.ds(ss * SLAB, SLAB)])


def _bn(x, g, b):
    m = jnp.mean(x, axis=0)
    v = jnp.var(x, axis=0)
    return (x - m) / jnp.sqrt(v + 1e-5) * g + b


def _dis_from(deg2_ref):
    deg = deg2_ref[0, :N] + deg2_ref[1, :N] + 1.0
    return lax.rsqrt(deg)


def _dense1_body(x0_ref, w_ref, b_ref, g_ref, be_ref, cw_ref, deg2_ref, out_ref):
    h = jnp.maximum(jnp.dot(x0_ref[...], w_ref[...],
                            preferred_element_type=jnp.float32) + b_ref[...], 0.0)
    h = _bn(h, g_ref[...], be_ref[...])
    gmat = jnp.dot(h, cw_ref[...], preferred_element_type=jnp.float32)
    out_ref[:N] = _dis_from(deg2_ref)[:, None] * gmat
    out_ref[N:] = jnp.zeros((NPAD - N, D), jnp.float32)


def _dense2_body(acc_ref, gp_ref, deg2_ref, cb_ref, ng_ref, nb_ref,
                 w_ref, b_ref, g_ref, be_ref, cw_ref, out_ref):
    dis = _dis_from(deg2_ref)
    x = dis[:, None] * (acc_ref[0, :N] + acc_ref[1, :N] + gp_ref[:N]) + cb_ref[...]
    x = jnp.maximum(_bn(x, ng_ref[...], nb_ref[...]), 0.0)
    h = jnp.maximum(jnp.dot(x, w_ref[...],
                            preferred_element_type=jnp.float32) + b_ref[...], 0.0)
    h = _bn(h, g_ref[...], be_ref[...])
    gmat = jnp.dot(h, cw_ref[...], preferred_element_type=jnp.float32)
    out_ref[:N] = dis[:, None] * gmat
    out_ref[N:] = jnp.zeros((NPAD - N, D), jnp.float32)


def _dense3_body(acc_ref, gp_ref, deg2_ref, cb_ref, ng_ref, nb_ref, batch_ref,
                 f1w_ref, f1b_ref, f2w_ref, f2b_ref, out_ref):
    dis = _dis_from(deg2_ref)
    x = dis[:, None] * (acc_ref[0, :N] + acc_ref[1, :N] + gp_ref[:N]) + cb_ref[...]
    x = jnp.maximum(_bn(x, ng_ref[...], nb_ref[...]), 0.0)
    onehot = (batch_ref[...][:, None]
              == lax.broadcasted_iota(jnp.int32, (N, G), 1)).astype(jnp.float32)
    sums = lax.dot_general(onehot, x, (((0,), (0,)), ((), ())),
                           preferred_element_type=jnp.float32)
    cnt = jnp.sum(onehot, axis=0)
    pooled = sums / jnp.maximum(cnt, 1.0)[:, None]
    y = jnp.maximum(jnp.dot(pooled, f1w_ref[...],
                            preferred_element_type=jnp.float32) + f1b_ref[...], 0.0)
    out_ref[...] = jnp.dot(y, f2w_ref[...],
                           preferred_element_type=jnp.float32) + f2b_ref[...]


def _tc_call(body, out_shape, *args):
    return pl.pallas_call(body, out_shape=out_shape)(*args)


@functools.lru_cache(maxsize=None)
def _sc_kernels():
    mesh = plsc.VectorSubcoreMesh(core_axis_name="c", subcore_axis_name="s",
                                  num_cores=NC, num_subcores=NS)
    params = pltpu.CompilerParams()
    deg_k = pl.kernel(
        _deg_body,
        out_type=jax.ShapeDtypeStruct((NC, NPAD), jnp.float32),
        mesh=mesh,
        compiler_params=params,
        scratch_types=[
            pltpu.VMEM((KCH, CHUNK), jnp.int32),
            pltpu.VMEM((CHUNK,), jnp.float32),
            pltpu.VMEM_SHARED((NPAD,), jnp.float32),
            pltpu.SemaphoreType.DMA,
        ],
    )
    mp_k = pl.kernel(
        _mp_body,
        out_type=jax.ShapeDtypeStruct((NC, NPAD, D), jnp.float32),
        mesh=mesh,
        compiler_params=params,
        scratch_types=[
            pltpu.VMEM((KCH, CHUNK), jnp.int32),
            pltpu.VMEM((MBLK, CHUNK), jnp.int32),
            pltpu.VMEM((MBLK, CHUNK), jnp.int32),
            pltpu.VMEM((CHUNK, D), jnp.float32),
            pltpu.VMEM((CHUNK, D), jnp.float32),
            pltpu.VMEM_SHARED((NPAD, D), jnp.float32),
            pltpu.SemaphoreType.DMA,
            pltpu.SemaphoreType.DMA,
        ],
    )
    return deg_k, mp_k


def kernel(x0, edge_index, batch,
           mlp0_W, mlp0_b, mlp0_gamma, mlp0_beta,
           conv0_W, conv0_b, norm0_gamma, norm0_beta,
           mlp1_W, mlp1_b, mlp1_gamma, mlp1_beta,
           conv1_W, conv1_b, norm1_gamma, norm1_beta,
           fc1_W, fc1_b, fc2_W, fc2_b):
    padv = N + (jnp.arange(EPAD - E, dtype=jnp.int32) % NPADROWS)
    r2 = jnp.concatenate([edge_index[0], padv]).reshape(EPAD // CHUNK, CHUNK)
    c2 = jnp.concatenate([edge_index[1], padv]).reshape(EPAD // CHUNK, CHUNK)
    ones_chunk = jnp.ones((CHUNK,), jnp.float32)
    zrow = jnp.zeros((SLAB,), jnp.float32)
    zslab = jnp.zeros((SLAB, D), jnp.float32)

    deg_kernel, mp_kernel = _sc_kernels()
    deg2 = deg_kernel(c2, ones_chunk, zrow)

    g0p = _tc_call(_dense1_body, jax.ShapeDtypeStruct((NPAD, D), jnp.float32),
                   x0, mlp0_W, mlp0_b, mlp0_gamma, mlp0_beta, conv0_W, deg2)
    acc0 = mp_kernel(g0p, r2, c2, zslab)
    g1p = _tc_call(_dense2_body, jax.ShapeDtypeStruct((NPAD, D), jnp.float32),
                   acc0, g0p, deg2, conv0_b, norm0_gamma, norm0_beta,
                   mlp1_W, mlp1_b, mlp1_gamma, mlp1_beta, conv1_W)
    acc1 = mp_kernel(g1p, r2, c2, zslab)
    y = _tc_call(_dense3_body, jax.ShapeDtypeStruct((G, NCLASSES_OUT), jnp.float32),
                 acc1, g1p, deg2, conv1_b, norm1_gamma, norm1_beta, batch,
                 fc1_W, fc1_b, fc2_W, fc2_b)
    return (y, y)
